# baseline (speedup 1.0000x reference)
import jax
import jax.numpy as jnp
from jax.experimental import pallas as pl


def _bcast_add_kernel(w_ref, xt_ref, o_ref):
    w = w_ref[...]
    xt = xt_ref[...]
    o_ref[...] = w[:, :, None] + xt[None, :, :]


def kernel(x, pos_embed_weight):
    seq_len, batch_size = x.shape
    _, dim = pos_embed_weight.shape

    w = pos_embed_weight[:seq_len]
    xt = x.reshape(1, seq_len)

    BI = 128
    out3 = pl.pallas_call(
        _bcast_add_kernel,
        grid=(seq_len // BI,),
        in_specs=[
            pl.BlockSpec((BI, dim), lambda i: (i, 0)),
            pl.BlockSpec((1, seq_len), lambda i: (0, 0)),
        ],
        out_specs=pl.BlockSpec((BI, dim, seq_len), lambda i: (i, 0, 0)),
        out_shape=jax.ShapeDtypeStruct((seq_len, dim, seq_len), jnp.float32),
    )(w, xt)

    return jnp.transpose(out3, (0, 2, 1))


# speedup vs baseline: 1.0330x; 1.0330x over previous
import jax
import jax.numpy as jnp
from jax.experimental import pallas as pl


def _bcast_add_kernel(w_ref, xt_ref, o_ref):
    w = w_ref[...]
    xt = xt_ref[...]
    o_ref[...] = w[:, :, None] + xt[None, :, :]


def kernel(x, pos_embed_weight):
    seq_len, batch_size = x.shape
    _, dim = pos_embed_weight.shape

    w = pos_embed_weight[:seq_len]
    xt = x.reshape(1, seq_len)

    BI = 32
    out3 = pl.pallas_call(
        _bcast_add_kernel,
        grid=(seq_len // BI,),
        in_specs=[
            pl.BlockSpec((BI, dim), lambda i: (i, 0)),
            pl.BlockSpec((1, seq_len), lambda i: (0, 0)),
        ],
        out_specs=pl.BlockSpec((BI, dim, seq_len), lambda i: (i, 0, 0)),
        out_shape=jax.ShapeDtypeStruct((seq_len, dim, seq_len), jnp.float32),
    )(w, xt)

    return jnp.transpose(out3, (0, 2, 1))
